# baseline (speedup 1.0000x reference)
import functools

import jax
import jax.numpy as jnp
from jax import lax
from jax.experimental import pallas as pl
from jax.experimental.pallas import tpu as pltpu
from jax.experimental.pallas import tpu_sc as plsc

_CHUNK = 128


def _mask_body(t_ref, m_ref):
    m_ref[...] = t_ref[...] == 0


def kernel(term, table):
    B, H = term.shape
    V, D = table.shape
    N = B * H

    info = plsc.get_sparse_core_info()
    NC, NS = info.num_cores, info.num_subcores
    NW = NC * NS
    per_w = N // NW
    n_ch = per_w // _CHUNK
    assert per_w * NW == N and n_ch * _CHUNK == per_w

    term_blocks = term.reshape(NW, n_ch, _CHUNK)
    mesh = plsc.VectorSubcoreMesh(core_axis_name="c", subcore_axis_name="s")

    @functools.partial(
        pl.kernel,
        mesh=mesh,
        compiler_params=pltpu.CompilerParams(use_tc_tiling_on_sc=False),
        out_type=jax.ShapeDtypeStruct((NW, n_ch, _CHUNK, D), jnp.float32),
        scratch_types=[
            pltpu.VMEM((n_ch, _CHUNK), jnp.int32),
            pltpu.VMEM((2, _CHUNK, D), jnp.float32),
            pltpu.SemaphoreType.DMA,
            pltpu.SemaphoreType.DMA,
        ],
    )
    def gather_k(term_hbm, table_hbm, out_hbm, idx_v, rows_v, g_sem, s_sem):
        wid = lax.axis_index("s") * NC + lax.axis_index("c")
        pltpu.sync_copy(term_hbm.at[wid], idx_v)

        pltpu.async_copy(table_hbm.at[idx_v.at[0]], rows_v.at[0], g_sem)

        def step(j, carry):
            slot = lax.rem(j, 2)
            nxt = lax.rem(j + 1, 2)

            @pl.when(j + 1 < n_ch)
            def _():
                pltpu.async_copy(
                    table_hbm.at[idx_v.at[j + 1]], rows_v.at[nxt], g_sem
                )

            pltpu.make_async_copy(
                table_hbm.at[idx_v.at[j]], rows_v.at[slot], g_sem
            ).wait()

            @pl.when(j > 0)
            def _():
                pltpu.make_async_copy(
                    rows_v.at[nxt], out_hbm.at[wid, j - 1], s_sem
                ).wait()

            pltpu.async_copy(rows_v.at[slot], out_hbm.at[wid, j], s_sem)
            return carry

        lax.fori_loop(0, n_ch, step, 0)
        pltpu.make_async_copy(
            rows_v.at[lax.rem(n_ch - 1, 2)], out_hbm.at[wid, n_ch - 1], s_sem
        ).wait()

    emb = gather_k(term_blocks, table).reshape(B, H, D)

    mask = pl.pallas_call(
        _mask_body,
        out_shape=jax.ShapeDtypeStruct((B, H), jnp.bool_),
    )(term)
    return emb, mask
